# baseline (speedup 1.0000x reference)
import functools

import jax
import jax.numpy as jnp
from jax import lax
from jax.experimental import pallas as pl
from jax.experimental.pallas import tpu as pltpu
from jax.experimental.pallas import tpu_sc as plsc

N = 10000
H = 128
E = 320000
NC = 2
NS = 16
NW = NC * NS
CH = 128
GPT = 80
EPT = GPT * CH
EPAD = NW * EPT
NPAD = EPAD - E
NP = 10240
RPT = NP // NS
ZR = 32
SGC = 16
NSTG = GPT // SGC
NCHUNK = EPAD // CH
CA = 0
CB = NCHUNK // NS - CA
EPS = 1e-5


def _deg_body(dst_hbm, out_hbm, dst_v, ones_v, zb_v, acc):
    cid = lax.axis_index("c")
    sid = lax.axis_index("s")
    wid = sid * NC + cid
    one16 = jnp.ones((16,), jnp.float32)
    zero16 = jnp.zeros((16,), jnp.float32)
    for i in range(CH // 16):
        ones_v[pl.ds(i * 16, 16)] = one16
    for i in range(640 // 16):
        zb_v[pl.ds(i * 16, 16)] = zero16

    pltpu.sync_copy(zb_v, acc.at[pl.ds(sid * 640, 640)])
    plsc.subcore_barrier()

    pltpu.sync_copy(dst_hbm.at[pl.ds(wid * GPT, GPT)], dst_v)

    def body(g, carry):
        pltpu.sync_copy(ones_v, acc.at[dst_v.at[g]], add=True)
        return carry

    lax.fori_loop(0, GPT, body, 0)
    plsc.subcore_barrier()

    pltpu.sync_copy(acc.at[pl.ds(sid * 640, 640)], zb_v)
    pltpu.sync_copy(zb_v, out_hbm.at[cid, pl.ds(sid * 640, 640)])


def _seg_body(y_hbm, src_hbm, dst_hbm, out_hbm,
              src_v, dst_v, rows_v, zb_v, acc, sem_a, sem_b):
    cid = lax.axis_index("c")
    sid = lax.axis_index("s")
    wid = sid * NC + cid
    zero16 = jnp.zeros((16,), jnp.float32)

    def zfill(r, carry):
        for c in range(H // 16):
            zb_v[r, pl.ds(c * 16, 16)] = zero16
        return carry

    lax.fori_loop(0, ZR, zfill, 0)

    def zbody(k, carry):
        pltpu.sync_copy(zb_v, acc.at[pl.ds(sid * RPT + k * ZR, ZR)])
        return carry

    lax.fori_loop(0, RPT // ZR, zbody, 0)
    plsc.subcore_barrier()

    def run_chunks(base, nstages):
        def stage_body(st, carry):
            off = base + st * SGC
            pltpu.sync_copy(src_hbm.at[pl.ds(off, SGC)], src_v)
            pltpu.sync_copy(dst_hbm.at[pl.ds(off, SGC)], dst_v)
            pltpu.async_copy(y_hbm.at[src_v.at[0]], rows_v.at[0], sem_a)

            def body(j, c2):
                g0 = 2 * j
                pltpu.async_copy(y_hbm.at[src_v.at[g0 + 1]], rows_v.at[1],
                                 sem_b)
                pltpu.make_async_copy(y_hbm.at[src_v.at[g0]], rows_v.at[0],
                                      sem_a).wait()
                pltpu.sync_copy(rows_v.at[0], acc.at[dst_v.at[g0]], add=True)

                @pl.when(j < SGC // 2 - 1)
                def _():
                    pltpu.async_copy(y_hbm.at[src_v.at[g0 + 2]],
                                     rows_v.at[0], sem_a)

                pltpu.make_async_copy(y_hbm.at[src_v.at[g0 + 1]],
                                      rows_v.at[1], sem_b).wait()
                pltpu.sync_copy(rows_v.at[1], acc.at[dst_v.at[g0 + 1]],
                                add=True)
                return c2

            lax.fori_loop(0, SGC // 2, body, 0)
            return carry

        lax.fori_loop(0, nstages, stage_body, 0)

    @pl.when(cid == 0)
    def _():
        run_chunks(sid * CA, CA // SGC)

    @pl.when(cid == 1)
    def _():
        run_chunks(NS * CA + sid * CB, CB // SGC)

    plsc.subcore_barrier()

    def wbody(k, carry):
        base = sid * RPT + k * ZR
        pltpu.sync_copy(acc.at[pl.ds(base, ZR)], zb_v)
        pltpu.sync_copy(zb_v, out_hbm.at[cid, pl.ds(base, ZR)])
        return carry

    lax.fori_loop(0, RPT // ZR, wbody, 0)


@functools.lru_cache(maxsize=None)
def _sc_kernels():
    mesh = plsc.VectorSubcoreMesh(core_axis_name="c", subcore_axis_name="s",
                                  num_cores=NC, num_subcores=NS)
    deg_kernel = pl.kernel(
        _deg_body,
        out_type=jax.ShapeDtypeStruct((NC, NP), jnp.float32),
        mesh=mesh,
        scratch_types=[
            pltpu.VMEM((GPT, CH), jnp.int32),
            pltpu.VMEM((CH,), jnp.float32),
            pltpu.VMEM((640,), jnp.float32),
            pltpu.VMEM_SHARED((NP,), jnp.float32),
        ],
    )
    seg_kernel = pl.kernel(
        _seg_body,
        out_type=jax.ShapeDtypeStruct((NC, NP, H), jnp.float32),
        mesh=mesh,
        scratch_types=[
            pltpu.VMEM((SGC, CH), jnp.int32),
            pltpu.VMEM((SGC, CH), jnp.int32),
            pltpu.VMEM((2, CH, H), jnp.float32),
            pltpu.VMEM((ZR, H), jnp.float32),
            pltpu.VMEM_SHARED((NP, H), jnp.float32),
            pltpu.SemaphoreType.DMA,
            pltpu.SemaphoreType.DMA,
        ],
    )
    return deg_kernel, seg_kernel


def _pre_body(degT_ref, x_ref, w_ref, y_ref, dinv_ref):
    dp = degT_ref[...]
    deg = dp[:, 0:1] + dp[:, 1:2] + 1.0
    dinv = lax.rsqrt(deg)
    dinv_ref[...] = dinv
    y_ref[...] = dinv * jnp.dot(x_ref[...], w_ref[...],
                                preferred_element_type=jnp.float32)


_pre_call = pl.pallas_call(
    _pre_body,
    out_shape=(jax.ShapeDtypeStruct((N, H), jnp.float32),
               jax.ShapeDtypeStruct((N, 1), jnp.float32)),
)


def _mid_body(has_res, has_next, *refs):
    if has_res:
        sp_ref, y_ref, dinv_ref, xprev_ref = refs[:4]
        rest = refs[4:]
    else:
        sp_ref, y_ref, dinv_ref = refs[:3]
        rest = refs[3:]
    if has_next:
        b_ref, g_ref, be_ref, wn_ref, outx_ref, outy_ref = rest
    else:
        b_ref, g_ref, be_ref, outx_ref = rest

    y = y_ref[...]
    s = sp_ref[0, :N, :] + sp_ref[1, :N, :]
    dinv = dinv_ref[...]
    t = dinv * (s + y) + b_ref[...]
    mu = jnp.mean(t, axis=0, keepdims=True)
    d = t - mu
    var = jnp.mean(d * d, axis=0, keepdims=True)
    x = g_ref[...] * d * lax.rsqrt(var + EPS) + be_ref[...]
    x = jnp.maximum(x, 0.0)
    if has_res:
        x = x + xprev_ref[...]
    outx_ref[...] = x
    if has_next:
        outy_ref[...] = dinv * jnp.dot(x, wn_ref[...],
                                       preferred_element_type=jnp.float32)


_mid_first = pl.pallas_call(
    functools.partial(_mid_body, False, True),
    out_shape=(jax.ShapeDtypeStruct((N, H), jnp.float32),
               jax.ShapeDtypeStruct((N, H), jnp.float32)),
)
_mid_res = pl.pallas_call(
    functools.partial(_mid_body, True, True),
    out_shape=(jax.ShapeDtypeStruct((N, H), jnp.float32),
               jax.ShapeDtypeStruct((N, H), jnp.float32)),
)
_mid_last = pl.pallas_call(
    functools.partial(_mid_body, True, False),
    out_shape=jax.ShapeDtypeStruct((N, H), jnp.float32),
)


def kernel(edge_index, emb, W1, b1, g1, be1, W2, b2, g2, be2, W3, b3, g3, be3):
    pad_dst = (jnp.arange(NPAD, dtype=jnp.int32) % (NP - N)) + N
    src_p = jnp.pad(edge_index[0], (0, NPAD)).reshape(NCHUNK, CH)
    dst_p = jnp.concatenate([edge_index[1], pad_dst]).reshape(NCHUNK, CH)
    b1r, g1r, be1r = b1.reshape(1, H), g1.reshape(1, H), be1.reshape(1, H)
    b2r, g2r, be2r = b2.reshape(1, H), g2.reshape(1, H), be2.reshape(1, H)
    b3r, g3r, be3r = b3.reshape(1, H), g3.reshape(1, H), be3.reshape(1, H)

    deg_kernel, seg_kernel = _sc_kernels()
    degp = deg_kernel(dst_p)[:, :N]
    y1, dinv = _pre_call(degp.T, emb, W1)
    s1 = seg_kernel(y1, src_p, dst_p)
    x2, y2 = _mid_first(s1, y1, dinv, b1r, g1r, be1r, W2)
    s2 = seg_kernel(y2, src_p, dst_p)
    x3, y3 = _mid_res(s2, y2, dinv, x2, b2r, g2r, be2r, W3)
    s3 = seg_kernel(y3, src_p, dst_p)
    return _mid_last(s3, y3, dinv, x3, b3r, g3r, be3r)


# speedup vs baseline: 1.1318x; 1.1318x over previous
import functools

import jax
import jax.numpy as jnp
from jax import lax
from jax.experimental import pallas as pl
from jax.experimental.pallas import tpu as pltpu
from jax.experimental.pallas import tpu_sc as plsc

N = 10000
H = 128
E = 320000
NC = 2
NS = 16
NW = NC * NS
CH = 128
GPT = 80
EPT = GPT * CH
EPAD = NW * EPT
NPAD = EPAD - E
NP = 10240
RPT = NP // NS
ZR = 32
SGC = 16
NSTG = GPT // SGC
NCHUNK = EPAD // CH
CA = 80
CB = NCHUNK // NS - CA
HH = H // 2
EPS = 1e-5


def _deg_body(dst_hbm, out_hbm, dst_v, ones_v, zb_v, acc):
    cid = lax.axis_index("c")
    sid = lax.axis_index("s")
    wid = sid * NC + cid
    one16 = jnp.ones((16,), jnp.float32)
    zero16 = jnp.zeros((16,), jnp.float32)
    for i in range(CH // 16):
        ones_v[pl.ds(i * 16, 16)] = one16
    for i in range(640 // 16):
        zb_v[pl.ds(i * 16, 16)] = zero16

    pltpu.sync_copy(zb_v, acc.at[pl.ds(sid * 640, 640)])
    plsc.subcore_barrier()

    pltpu.sync_copy(dst_hbm.at[pl.ds(wid * GPT, GPT)], dst_v)

    def body(g, carry):
        pltpu.sync_copy(ones_v, acc.at[dst_v.at[g]], add=True)
        return carry

    lax.fori_loop(0, GPT, body, 0)
    plsc.subcore_barrier()

    pltpu.sync_copy(acc.at[pl.ds(sid * 640, 640)], zb_v)
    pltpu.sync_copy(zb_v, out_hbm.at[cid, pl.ds(sid * 640, 640)])


def _seg_body(y_hbm, src_hbm, dst_hbm, out_hbm,
              src_v, dst_v, rows_v, zb_v, acc, sem_a, sem_b):
    cid = lax.axis_index("c")
    sid = lax.axis_index("s")
    zero16 = jnp.zeros((16,), jnp.float32)

    def zfill(r, carry):
        for c in range(H // 16):
            zb_v[r, pl.ds(c * 16, 16)] = zero16
        return carry

    lax.fori_loop(0, ZR, zfill, 0)

    def zbody(k, carry):
        pltpu.sync_copy(zb_v, acc.at[pl.ds(sid * RPT + k * ZR, ZR)])
        return carry

    lax.fori_loop(0, RPT // ZR, zbody, 0)
    plsc.subcore_barrier()

    def run_chunks(base, nstages):
        def stage_body(st, carry):
            off = base + st * SGC
            pltpu.sync_copy(src_hbm.at[pl.ds(off, SGC)], src_v)
            pltpu.sync_copy(dst_hbm.at[pl.ds(off, SGC)], dst_v)
            pltpu.async_copy(y_hbm.at[src_v.at[0]], rows_v.at[0], sem_a)

            def body(j, c2):
                g0 = 2 * j
                pltpu.async_copy(y_hbm.at[src_v.at[g0 + 1]], rows_v.at[1],
                                 sem_b)
                pltpu.make_async_copy(y_hbm.at[src_v.at[g0]], rows_v.at[0],
                                      sem_a).wait()
                pltpu.sync_copy(rows_v.at[0], acc.at[dst_v.at[g0]], add=True)

                @pl.when(j < SGC // 2 - 1)
                def _():
                    pltpu.async_copy(y_hbm.at[src_v.at[g0 + 2]],
                                     rows_v.at[0], sem_a)

                pltpu.make_async_copy(y_hbm.at[src_v.at[g0 + 1]],
                                      rows_v.at[1], sem_b).wait()
                pltpu.sync_copy(rows_v.at[1], acc.at[dst_v.at[g0 + 1]],
                                add=True)
                return c2

            lax.fori_loop(0, SGC // 2, body, 0)
            return carry

        lax.fori_loop(0, nstages, stage_body, 0)

    @pl.when(cid == 0)
    def _():
        run_chunks(sid * CA, CA // SGC)

    @pl.when(cid == 1)
    def _():
        run_chunks(NS * CA + sid * CB, CB // SGC)

    plsc.subcore_barrier()

    def wbody(k, carry):
        base = sid * RPT + k * ZR
        pltpu.sync_copy(acc.at[pl.ds(base, ZR)], zb_v)
        pltpu.sync_copy(zb_v, out_hbm.at[cid, pl.ds(base, ZR)])
        return carry

    lax.fori_loop(0, RPT // ZR, wbody, 0)


@functools.lru_cache(maxsize=None)
def _sc_kernels():
    mesh = plsc.VectorSubcoreMesh(core_axis_name="c", subcore_axis_name="s",
                                  num_cores=NC, num_subcores=NS)
    deg_kernel = pl.kernel(
        _deg_body,
        out_type=jax.ShapeDtypeStruct((NC, NP), jnp.float32),
        mesh=mesh,
        scratch_types=[
            pltpu.VMEM((GPT, CH), jnp.int32),
            pltpu.VMEM((CH,), jnp.float32),
            pltpu.VMEM((640,), jnp.float32),
            pltpu.VMEM_SHARED((NP,), jnp.float32),
        ],
    )
    seg_kernel = pl.kernel(
        _seg_body,
        out_type=jax.ShapeDtypeStruct((NC, NP, H), jnp.float32),
        mesh=mesh,
        scratch_types=[
            pltpu.VMEM((SGC, CH), jnp.int32),
            pltpu.VMEM((SGC, CH), jnp.int32),
            pltpu.VMEM((2, CH, H), jnp.float32),
            pltpu.VMEM((ZR, H), jnp.float32),
            pltpu.VMEM_SHARED((NP, H), jnp.float32),
            pltpu.SemaphoreType.DMA,
            pltpu.SemaphoreType.DMA,
        ],
    )
    return deg_kernel, seg_kernel


def _pre_body(degT_ref, x_ref, w_ref, y_ref, dinv_ref):
    dp = degT_ref[...]
    deg = dp[:, 0:1] + dp[:, 1:2] + 1.0
    dinv = lax.rsqrt(deg)
    dinv_ref[...] = dinv
    y_ref[...] = dinv * jnp.dot(x_ref[...], w_ref[...],
                                preferred_element_type=jnp.float32)


_pre_call = pl.pallas_call(
    _pre_body,
    out_shape=(jax.ShapeDtypeStruct((N, H), jnp.float32),
               jax.ShapeDtypeStruct((N, 1), jnp.float32)),
)


def _mid_body(has_res, has_next, *refs):
    if has_res:
        sp_ref, y_ref, dinv_ref, xprev_ref = refs[:4]
        rest = refs[4:]
    else:
        sp_ref, y_ref, dinv_ref = refs[:3]
        rest = refs[3:]
    if has_next:
        b_ref, g_ref, be_ref, wn_ref, outx_ref, outy_ref = rest
    else:
        b_ref, g_ref, be_ref, outx_ref = rest

    y = y_ref[...]
    s = sp_ref[0, :N, :] + sp_ref[1, :N, :]
    dinv = dinv_ref[...]
    t = dinv * (s + y) + b_ref[...]
    mu = jnp.mean(t, axis=0, keepdims=True)
    d = t - mu
    var = jnp.mean(d * d, axis=0, keepdims=True)
    x = g_ref[...] * d * lax.rsqrt(var + EPS) + be_ref[...]
    x = jnp.maximum(x, 0.0)
    if has_res:
        x = x + xprev_ref[...]
    outx_ref[...] = x
    if has_next:
        outy_ref[...] = dinv * jnp.dot(x, wn_ref[...],
                                       preferred_element_type=jnp.float32)


_mid_first = pl.pallas_call(
    functools.partial(_mid_body, False, True),
    out_shape=(jax.ShapeDtypeStruct((N, H), jnp.float32),
               jax.ShapeDtypeStruct((N, H), jnp.float32)),
)
_mid_res = pl.pallas_call(
    functools.partial(_mid_body, True, True),
    out_shape=(jax.ShapeDtypeStruct((N, H), jnp.float32),
               jax.ShapeDtypeStruct((N, H), jnp.float32)),
)
_mid_last = pl.pallas_call(
    functools.partial(_mid_body, True, False),
    out_shape=jax.ShapeDtypeStruct((N, H), jnp.float32),
)


def kernel(edge_index, emb, W1, b1, g1, be1, W2, b2, g2, be2, W3, b3, g3, be3):
    pad_dst = (jnp.arange(NPAD, dtype=jnp.int32) % (NP - N)) + N
    src_p = jnp.pad(edge_index[0], (0, NPAD)).reshape(NCHUNK, CH)
    dst_p = jnp.concatenate([edge_index[1], pad_dst]).reshape(NCHUNK, CH)
    b1r, g1r, be1r = b1.reshape(1, H), g1.reshape(1, H), be1.reshape(1, H)
    b2r, g2r, be2r = b2.reshape(1, H), g2.reshape(1, H), be2.reshape(1, H)
    b3r, g3r, be3r = b3.reshape(1, H), g3.reshape(1, H), be3.reshape(1, H)

    deg_kernel, seg_kernel = _sc_kernels()
    degp = deg_kernel(dst_p)[:, :N]
    y1, dinv = _pre_call(degp.T, emb, W1)
    s1 = seg_kernel(y1, src_p, dst_p)
    x2, y2 = _mid_first(s1, y1, dinv, b1r, g1r, be1r, W2)
    s2 = seg_kernel(y2, src_p, dst_p)
    x3, y3 = _mid_res(s2, y2, dinv, x2, b2r, g2r, be2r, W3)
    s3 = seg_kernel(y3, src_p, dst_p)
    return _mid_last(s3, y3, dinv, x3, b3r, g3r, be3r)


# speedup vs baseline: 1.1336x; 1.0015x over previous
import functools

import jax
import jax.numpy as jnp
from jax import lax
from jax.experimental import pallas as pl
from jax.experimental.pallas import tpu as pltpu
from jax.experimental.pallas import tpu_sc as plsc

N = 10000
H = 128
E = 320000
NC = 2
NS = 16
NW = NC * NS
CH = 128
GPT = 80
EPT = GPT * CH
EPAD = NW * EPT
NPAD = EPAD - E
NP = 10240
RPT = NP // NS
ZR = 32
SGC = 16
NSTG = GPT // SGC
NCHUNK = EPAD // CH
CA = 80
CB = NCHUNK // NS - CA
HH = H // 2
EPS = 1e-5


def _deg_body(dst_hbm, out_hbm, dst_v, ones_v, zb_v, acc):
    cid = lax.axis_index("c")
    sid = lax.axis_index("s")
    wid = sid * NC + cid
    one16 = jnp.ones((16,), jnp.float32)
    zero16 = jnp.zeros((16,), jnp.float32)
    for i in range(CH // 16):
        ones_v[pl.ds(i * 16, 16)] = one16
    for i in range(640 // 16):
        zb_v[pl.ds(i * 16, 16)] = zero16

    pltpu.sync_copy(zb_v, acc.at[pl.ds(sid * 640, 640)])
    plsc.subcore_barrier()

    pltpu.sync_copy(dst_hbm.at[pl.ds(wid * GPT, GPT)], dst_v)

    def body(g, carry):
        pltpu.sync_copy(ones_v, acc.at[dst_v.at[g]], add=True)
        return carry

    lax.fori_loop(0, GPT, body, 0)
    plsc.subcore_barrier()

    pltpu.sync_copy(acc.at[pl.ds(sid * 640, 640)], zb_v)
    pltpu.sync_copy(zb_v, out_hbm.at[cid, pl.ds(sid * 640, 640)])


def _seg_body(y_hbm, src_hbm, dst_hbm, out_hbm,
              src_v, dst_v, rows_v, zb_v, acc, sem_a, sem_b):
    cid = lax.axis_index("c")
    sid = lax.axis_index("s")
    zero16 = jnp.zeros((16,), jnp.float32)

    def zfill(r, carry):
        for c in range(H // 16):
            zb_v[r, pl.ds(c * 16, 16)] = zero16
        return carry

    lax.fori_loop(0, ZR, zfill, 0)

    def zbody(k, carry):
        pltpu.sync_copy(zb_v, acc.at[pl.ds(sid * RPT + k * ZR, ZR)])
        return carry

    lax.fori_loop(0, RPT // ZR, zbody, 0)
    plsc.subcore_barrier()

    def run_chunks(base, nstages):
        def stage_body(st, carry):
            off = base + st * SGC
            pltpu.sync_copy(src_hbm.at[pl.ds(off, SGC)], src_v)
            pltpu.sync_copy(dst_hbm.at[pl.ds(off, SGC)], dst_v)
            pltpu.async_copy(y_hbm.at[src_v.at[0]], rows_v.at[0], sem_a)

            def body(j, c2):
                g0 = 2 * j
                pltpu.async_copy(y_hbm.at[src_v.at[g0 + 1]], rows_v.at[1],
                                 sem_b)
                pltpu.make_async_copy(y_hbm.at[src_v.at[g0]], rows_v.at[0],
                                      sem_a).wait()
                pltpu.sync_copy(rows_v.at[0], acc.at[dst_v.at[g0]], add=True)

                @pl.when(j < SGC // 2 - 1)
                def _():
                    pltpu.async_copy(y_hbm.at[src_v.at[g0 + 2]],
                                     rows_v.at[0], sem_a)

                pltpu.make_async_copy(y_hbm.at[src_v.at[g0 + 1]],
                                      rows_v.at[1], sem_b).wait()
                pltpu.sync_copy(rows_v.at[1], acc.at[dst_v.at[g0 + 1]],
                                add=True)
                return c2

            lax.fori_loop(0, SGC // 2, body, 0)
            return carry

        lax.fori_loop(0, nstages, stage_body, 0)

    wid = sid * NC + cid
    run_chunks(wid * GPT, NSTG)
    plsc.subcore_barrier()

    def wbody(k, carry):
        base = sid * RPT + k * ZR
        pltpu.sync_copy(acc.at[pl.ds(base, ZR)], zb_v)
        pltpu.sync_copy(zb_v, out_hbm.at[cid, pl.ds(base, ZR)])
        return carry

    lax.fori_loop(0, RPT // ZR, wbody, 0)


@functools.lru_cache(maxsize=None)
def _sc_kernels():
    mesh = plsc.VectorSubcoreMesh(core_axis_name="c", subcore_axis_name="s",
                                  num_cores=NC, num_subcores=NS)
    deg_kernel = pl.kernel(
        _deg_body,
        out_type=jax.ShapeDtypeStruct((NC, NP), jnp.float32),
        mesh=mesh,
        scratch_types=[
            pltpu.VMEM((GPT, CH), jnp.int32),
            pltpu.VMEM((CH,), jnp.float32),
            pltpu.VMEM((640,), jnp.float32),
            pltpu.VMEM_SHARED((NP,), jnp.float32),
        ],
    )
    seg_kernel = pl.kernel(
        _seg_body,
        out_type=jax.ShapeDtypeStruct((NC, NP, H), jnp.float32),
        mesh=mesh,
        scratch_types=[
            pltpu.VMEM((SGC, CH), jnp.int32),
            pltpu.VMEM((SGC, CH), jnp.int32),
            pltpu.VMEM((2, CH, H), jnp.float32),
            pltpu.VMEM((ZR, H), jnp.float32),
            pltpu.VMEM_SHARED((NP, H), jnp.float32),
            pltpu.SemaphoreType.DMA,
            pltpu.SemaphoreType.DMA,
        ],
    )
    return deg_kernel, seg_kernel


def _pre_body(degT_ref, x_ref, w_ref, y_ref, dinv_ref):
    dp = degT_ref[...]
    deg = dp[:, 0:1] + dp[:, 1:2] + 1.0
    dinv = lax.rsqrt(deg)
    dinv_ref[...] = dinv
    y_ref[...] = dinv * jnp.dot(x_ref[...], w_ref[...],
                                preferred_element_type=jnp.float32)


_pre_call = pl.pallas_call(
    _pre_body,
    out_shape=(jax.ShapeDtypeStruct((N, H), jnp.float32),
               jax.ShapeDtypeStruct((N, 1), jnp.float32)),
)


def _mid_body(has_res, has_next, *refs):
    if has_res:
        sp_ref, y_ref, dinv_ref, xprev_ref = refs[:4]
        rest = refs[4:]
    else:
        sp_ref, y_ref, dinv_ref = refs[:3]
        rest = refs[3:]
    if has_next:
        b_ref, g_ref, be_ref, wn_ref, outx_ref, outy_ref = rest
    else:
        b_ref, g_ref, be_ref, outx_ref = rest

    y = y_ref[...]
    s = sp_ref[0, :N, :] + sp_ref[1, :N, :]
    dinv = dinv_ref[...]
    t = dinv * (s + y) + b_ref[...]
    mu = jnp.mean(t, axis=0, keepdims=True)
    d = t - mu
    var = jnp.mean(d * d, axis=0, keepdims=True)
    x = g_ref[...] * d * lax.rsqrt(var + EPS) + be_ref[...]
    x = jnp.maximum(x, 0.0)
    if has_res:
        x = x + xprev_ref[...]
    outx_ref[...] = x
    if has_next:
        outy_ref[...] = dinv * jnp.dot(x, wn_ref[...],
                                       preferred_element_type=jnp.float32)


_mid_first = pl.pallas_call(
    functools.partial(_mid_body, False, True),
    out_shape=(jax.ShapeDtypeStruct((N, H), jnp.float32),
               jax.ShapeDtypeStruct((N, H), jnp.float32)),
)
_mid_res = pl.pallas_call(
    functools.partial(_mid_body, True, True),
    out_shape=(jax.ShapeDtypeStruct((N, H), jnp.float32),
               jax.ShapeDtypeStruct((N, H), jnp.float32)),
)
_mid_last = pl.pallas_call(
    functools.partial(_mid_body, True, False),
    out_shape=jax.ShapeDtypeStruct((N, H), jnp.float32),
)


def kernel(edge_index, emb, W1, b1, g1, be1, W2, b2, g2, be2, W3, b3, g3, be3):
    pad_dst = (jnp.arange(NPAD, dtype=jnp.int32) % (NP - N)) + N
    src_p = jnp.pad(edge_index[0], (0, NPAD)).reshape(NCHUNK, CH)
    dst_p = jnp.concatenate([edge_index[1], pad_dst]).reshape(NCHUNK, CH)
    b1r, g1r, be1r = b1.reshape(1, H), g1.reshape(1, H), be1.reshape(1, H)
    b2r, g2r, be2r = b2.reshape(1, H), g2.reshape(1, H), be2.reshape(1, H)
    b3r, g3r, be3r = b3.reshape(1, H), g3.reshape(1, H), be3.reshape(1, H)

    deg_kernel, seg_kernel = _sc_kernels()
    degp = deg_kernel(dst_p)[:, :N]
    y1, dinv = _pre_call(degp.T, emb, W1)
    s1 = seg_kernel(y1, src_p, dst_p)
    x2, y2 = _mid_first(s1, y1, dinv, b1r, g1r, be1r, W2)
    s2 = seg_kernel(y2, src_p, dst_p)
    x3, y3 = _mid_res(s2, y2, dinv, x2, b2r, g2r, be2r, W3)
    s3 = seg_kernel(y3, src_p, dst_p)
    return _mid_last(s3, y3, dinv, x3, b3r, g3r, be3r)


# speedup vs baseline: 1.2608x; 1.1122x over previous
import functools

import jax
import jax.numpy as jnp
from jax import lax
from jax.experimental import pallas as pl
from jax.experimental.pallas import tpu as pltpu
from jax.experimental.pallas import tpu_sc as plsc

N = 10000
H = 128
E = 320000
NC = 2
NS = 16
NW = NC * NS
CH = 128
GPT = 80
EPT = GPT * CH
EPAD = NW * EPT
NPAD = EPAD - E
NP = 10240
RPT = NP // NS
ZR = 32
SGC = 16
NSTG = GPT // SGC
NCHUNK = EPAD // CH
CA = 80
CB = NCHUNK // NS - CA
HH = H // 2
EPS = 1e-5


def _deg_body(dst_hbm, out_hbm, dst_v, ones_v, zb_v, acc):
    cid = lax.axis_index("c")
    sid = lax.axis_index("s")
    wid = sid * NC + cid
    one16 = jnp.ones((16,), jnp.float32)
    zero16 = jnp.zeros((16,), jnp.float32)
    for i in range(CH // 16):
        ones_v[pl.ds(i * 16, 16)] = one16
    for i in range(640 // 16):
        zb_v[pl.ds(i * 16, 16)] = zero16

    pltpu.sync_copy(zb_v, acc.at[pl.ds(sid * 640, 640)])
    plsc.subcore_barrier()

    pltpu.sync_copy(dst_hbm.at[wid], dst_v)

    def body(g, carry):
        pltpu.sync_copy(ones_v, acc.at[dst_v.at[g]], add=True)
        return carry

    lax.fori_loop(0, GPT, body, 0)
    plsc.subcore_barrier()

    pltpu.sync_copy(acc.at[pl.ds(sid * 640, 640)], zb_v)
    pltpu.sync_copy(zb_v, out_hbm.at[cid, pl.ds(sid * 640, 640)])


def _seg_body(y_hbm, src_hbm, dst_hbm, out_hbm,
              src_v, dst_v, rows_v, zb_v, acc, sem_a, sem_b):
    cid = lax.axis_index("c")
    sid = lax.axis_index("s")
    zero16 = jnp.zeros((16,), jnp.float32)

    def zfill(r, carry):
        for c in range(H // 16):
            zb_v[r, pl.ds(c * 16, 16)] = zero16
        return carry

    lax.fori_loop(0, ZR, zfill, 0)

    def zbody(k, carry):
        pltpu.sync_copy(zb_v, acc.at[pl.ds(sid * RPT + k * ZR, ZR)])
        return carry

    lax.fori_loop(0, RPT // ZR, zbody, 0)
    plsc.subcore_barrier()

    wid = sid * NC + cid

    def run_chunks(base, nstages):
        def stage_body(st, carry):
            pltpu.sync_copy(src_hbm.at[wid, pl.ds(st * SGC, SGC)], src_v)
            pltpu.sync_copy(dst_hbm.at[wid, pl.ds(st * SGC, SGC)], dst_v)
            pltpu.async_copy(y_hbm.at[src_v.at[0]], rows_v.at[0], sem_a)

            def body(j, c2):
                g0 = 2 * j
                pltpu.async_copy(y_hbm.at[src_v.at[g0 + 1]], rows_v.at[1],
                                 sem_b)
                pltpu.make_async_copy(y_hbm.at[src_v.at[g0]], rows_v.at[0],
                                      sem_a).wait()
                pltpu.sync_copy(rows_v.at[0], acc.at[dst_v.at[g0]], add=True)

                @pl.when(j < SGC // 2 - 1)
                def _():
                    pltpu.async_copy(y_hbm.at[src_v.at[g0 + 2]],
                                     rows_v.at[0], sem_a)

                pltpu.make_async_copy(y_hbm.at[src_v.at[g0 + 1]],
                                      rows_v.at[1], sem_b).wait()
                pltpu.sync_copy(rows_v.at[1], acc.at[dst_v.at[g0 + 1]],
                                add=True)
                return c2

            lax.fori_loop(0, SGC // 2, body, 0)
            return carry

        lax.fori_loop(0, nstages, stage_body, 0)

    run_chunks(0, NSTG)
    plsc.subcore_barrier()

    def wbody(k, carry):
        base = sid * RPT + k * ZR
        pltpu.sync_copy(acc.at[pl.ds(base, ZR)], zb_v)
        pltpu.sync_copy(zb_v, out_hbm.at[cid, pl.ds(base, ZR)])
        return carry

    lax.fori_loop(0, RPT // ZR, wbody, 0)


@functools.lru_cache(maxsize=None)
def _sc_kernels():
    mesh = plsc.VectorSubcoreMesh(core_axis_name="c", subcore_axis_name="s",
                                  num_cores=NC, num_subcores=NS)
    deg_kernel = pl.kernel(
        _deg_body,
        out_type=jax.ShapeDtypeStruct((NC, NP), jnp.float32),
        mesh=mesh,
        scratch_types=[
            pltpu.VMEM((GPT, CH), jnp.int32),
            pltpu.VMEM((CH,), jnp.float32),
            pltpu.VMEM((640,), jnp.float32),
            pltpu.VMEM_SHARED((NP,), jnp.float32),
        ],
    )
    seg_kernel = pl.kernel(
        _seg_body,
        out_type=jax.ShapeDtypeStruct((NC, NP, H), jnp.float32),
        mesh=mesh,
        scratch_types=[
            pltpu.VMEM((SGC, CH), jnp.int32),
            pltpu.VMEM((SGC, CH), jnp.int32),
            pltpu.VMEM((2, CH, H), jnp.float32),
            pltpu.VMEM((ZR, H), jnp.float32),
            pltpu.VMEM_SHARED((NP, H), jnp.float32),
            pltpu.SemaphoreType.DMA,
            pltpu.SemaphoreType.DMA,
        ],
    )
    return deg_kernel, seg_kernel


def _pre_body(degT_ref, x_ref, w_ref, y_ref, dinv_ref):
    dp = degT_ref[...]
    deg = dp[:, 0:1] + dp[:, 1:2] + 1.0
    dinv = lax.rsqrt(deg)
    dinv_ref[...] = dinv
    y_ref[...] = dinv * jnp.dot(x_ref[...], w_ref[...],
                                preferred_element_type=jnp.float32)


_pre_call = pl.pallas_call(
    _pre_body,
    out_shape=(jax.ShapeDtypeStruct((N, H), jnp.float32),
               jax.ShapeDtypeStruct((N, 1), jnp.float32)),
)


def _mid_body(has_res, has_next, *refs):
    if has_res:
        sp_ref, y_ref, dinv_ref, xprev_ref = refs[:4]
        rest = refs[4:]
    else:
        sp_ref, y_ref, dinv_ref = refs[:3]
        rest = refs[3:]
    if has_next:
        b_ref, g_ref, be_ref, wn_ref, outx_ref, outy_ref = rest
    else:
        b_ref, g_ref, be_ref, outx_ref = rest

    y = y_ref[...]
    s = sp_ref[0, :N, :] + sp_ref[1, :N, :]
    dinv = dinv_ref[...]
    t = dinv * (s + y) + b_ref[...]
    mu = jnp.mean(t, axis=0, keepdims=True)
    d = t - mu
    var = jnp.mean(d * d, axis=0, keepdims=True)
    x = g_ref[...] * d * lax.rsqrt(var + EPS) + be_ref[...]
    x = jnp.maximum(x, 0.0)
    if has_res:
        x = x + xprev_ref[...]
    outx_ref[...] = x
    if has_next:
        outy_ref[...] = dinv * jnp.dot(x, wn_ref[...],
                                       preferred_element_type=jnp.float32)


_mid_first = pl.pallas_call(
    functools.partial(_mid_body, False, True),
    out_shape=(jax.ShapeDtypeStruct((N, H), jnp.float32),
               jax.ShapeDtypeStruct((N, H), jnp.float32)),
)
_mid_res = pl.pallas_call(
    functools.partial(_mid_body, True, True),
    out_shape=(jax.ShapeDtypeStruct((N, H), jnp.float32),
               jax.ShapeDtypeStruct((N, H), jnp.float32)),
)
_mid_last = pl.pallas_call(
    functools.partial(_mid_body, True, False),
    out_shape=jax.ShapeDtypeStruct((N, H), jnp.float32),
)


def kernel(edge_index, emb, W1, b1, g1, be1, W2, b2, g2, be2, W3, b3, g3, be3):
    pad_dst = (jnp.arange(NPAD, dtype=jnp.int32) % (NP - N)) + N
    src_p = jnp.pad(edge_index[0], (0, NPAD)).reshape(NW, GPT, CH)
    dst_p = jnp.concatenate([edge_index[1], pad_dst]).reshape(NW, GPT, CH)
    b1r, g1r, be1r = b1.reshape(1, H), g1.reshape(1, H), be1.reshape(1, H)
    b2r, g2r, be2r = b2.reshape(1, H), g2.reshape(1, H), be2.reshape(1, H)
    b3r, g3r, be3r = b3.reshape(1, H), g3.reshape(1, H), be3.reshape(1, H)

    deg_kernel, seg_kernel = _sc_kernels()
    degp = deg_kernel(dst_p)[:, :N]
    y1, dinv = _pre_call(degp.T, emb, W1)
    s1 = seg_kernel(y1, src_p, dst_p)
    x2, y2 = _mid_first(s1, y1, dinv, b1r, g1r, be1r, W2)
    s2 = seg_kernel(y2, src_p, dst_p)
    x3, y3 = _mid_res(s2, y2, dinv, x2, b2r, g2r, be2r, W3)
    s3 = seg_kernel(y3, src_p, dst_p)
    return _mid_last(s3, y3, dinv, x3, b3r, g3r, be3r)


# speedup vs baseline: 1.2926x; 1.0252x over previous
import functools

import jax
import jax.numpy as jnp
from jax import lax
from jax.experimental import pallas as pl
from jax.experimental.pallas import tpu as pltpu
from jax.experimental.pallas import tpu_sc as plsc

N = 10000
H = 128
E = 320000
NC = 2
NS = 16
NW = NC * NS
CH = 128
GPT = 80
EPT = GPT * CH
EPAD = NW * EPT
NPAD = EPAD - E
NP = 10240
RPT = NP // NS
ZR = 32
SGC = 40
NSTG = GPT // SGC
NCHUNK = EPAD // CH
CA = 80
CB = NCHUNK // NS - CA
HH = H // 2
EPS = 1e-5


def _deg_body(dst_hbm, out_hbm, dst_v, ones_v, zb_v, acc):
    cid = lax.axis_index("c")
    sid = lax.axis_index("s")
    wid = sid * NC + cid
    one16 = jnp.ones((16,), jnp.float32)
    zero16 = jnp.zeros((16,), jnp.float32)
    for i in range(CH // 16):
        ones_v[pl.ds(i * 16, 16)] = one16
    for i in range(640 // 16):
        zb_v[pl.ds(i * 16, 16)] = zero16

    pltpu.sync_copy(zb_v, acc.at[pl.ds(sid * 640, 640)])
    plsc.subcore_barrier()

    pltpu.sync_copy(dst_hbm.at[wid], dst_v)

    def body(g, carry):
        pltpu.sync_copy(ones_v, acc.at[dst_v.at[g]], add=True)
        return carry

    lax.fori_loop(0, GPT, body, 0)
    plsc.subcore_barrier()

    pltpu.sync_copy(acc.at[pl.ds(sid * 640, 640)], zb_v)
    pltpu.sync_copy(zb_v, out_hbm.at[cid, pl.ds(sid * 640, 640)])


def _seg_body(y_hbm, src_hbm, dst_hbm, out_hbm,
              src_v, dst_v, rows_v, acc, sem_a, sem_b):
    cid = lax.axis_index("c")
    sid = lax.axis_index("s")
    zero16 = jnp.zeros((16,), jnp.float32)

    def zfill(r, carry):
        for c in range(H // 16):
            rows_v[0, r, pl.ds(c * 16, 16)] = zero16
        return carry

    lax.fori_loop(0, CH, zfill, 0)

    def zbody(k, carry):
        pltpu.sync_copy(rows_v.at[0], acc.at[pl.ds(sid * RPT + k * CH, CH)])
        return carry

    lax.fori_loop(0, RPT // CH, zbody, 0)
    plsc.subcore_barrier()

    wid = sid * NC + cid

    def run_chunks(base, nstages):
        def stage_body(st, carry):
            pltpu.sync_copy(src_hbm.at[wid, pl.ds(st * SGC, SGC)], src_v)
            pltpu.sync_copy(dst_hbm.at[wid, pl.ds(st * SGC, SGC)], dst_v)
            pltpu.async_copy(y_hbm.at[src_v.at[0]], rows_v.at[0], sem_a)

            def body(j, c2):
                g0 = 2 * j
                pltpu.async_copy(y_hbm.at[src_v.at[g0 + 1]], rows_v.at[1],
                                 sem_b)
                pltpu.make_async_copy(y_hbm.at[src_v.at[g0]], rows_v.at[0],
                                      sem_a).wait()
                pltpu.sync_copy(rows_v.at[0], acc.at[dst_v.at[g0]], add=True)

                @pl.when(j < SGC // 2 - 1)
                def _():
                    pltpu.async_copy(y_hbm.at[src_v.at[g0 + 2]],
                                     rows_v.at[0], sem_a)

                pltpu.make_async_copy(y_hbm.at[src_v.at[g0 + 1]],
                                      rows_v.at[1], sem_b).wait()
                pltpu.sync_copy(rows_v.at[1], acc.at[dst_v.at[g0 + 1]],
                                add=True)
                return c2

            lax.fori_loop(0, SGC // 2, body, 0)
            return carry

        lax.fori_loop(0, nstages, stage_body, 0)

    run_chunks(0, NSTG)
    plsc.subcore_barrier()

    def wbody(k, carry):
        base = sid * RPT + k * CH
        pltpu.sync_copy(acc.at[pl.ds(base, CH)], rows_v.at[0])
        pltpu.sync_copy(rows_v.at[0], out_hbm.at[cid, pl.ds(base, CH)])
        return carry

    lax.fori_loop(0, RPT // CH, wbody, 0)


@functools.lru_cache(maxsize=None)
def _sc_kernels():
    mesh = plsc.VectorSubcoreMesh(core_axis_name="c", subcore_axis_name="s",
                                  num_cores=NC, num_subcores=NS)
    deg_kernel = pl.kernel(
        _deg_body,
        out_type=jax.ShapeDtypeStruct((NC, NP), jnp.float32),
        mesh=mesh,
        scratch_types=[
            pltpu.VMEM((GPT, CH), jnp.int32),
            pltpu.VMEM((CH,), jnp.float32),
            pltpu.VMEM((640,), jnp.float32),
            pltpu.VMEM_SHARED((NP,), jnp.float32),
        ],
    )
    seg_kernel = pl.kernel(
        _seg_body,
        out_type=jax.ShapeDtypeStruct((NC, NP, H), jnp.float32),
        mesh=mesh,
        scratch_types=[
            pltpu.VMEM((SGC, CH), jnp.int32),
            pltpu.VMEM((SGC, CH), jnp.int32),
            pltpu.VMEM((2, CH, H), jnp.float32),
            pltpu.VMEM_SHARED((NP, H), jnp.float32),
            pltpu.SemaphoreType.DMA,
            pltpu.SemaphoreType.DMA,
        ],
    )
    return deg_kernel, seg_kernel


def _pre_body(degT_ref, x_ref, w_ref, y_ref, dinv_ref):
    dp = degT_ref[...]
    deg = dp[:, 0:1] + dp[:, 1:2] + 1.0
    dinv = lax.rsqrt(deg)
    dinv_ref[...] = dinv
    y_ref[...] = dinv * jnp.dot(x_ref[...], w_ref[...],
                                preferred_element_type=jnp.float32)


_pre_call = pl.pallas_call(
    _pre_body,
    out_shape=(jax.ShapeDtypeStruct((N, H), jnp.float32),
               jax.ShapeDtypeStruct((N, 1), jnp.float32)),
)


def _mid_body(has_res, has_next, *refs):
    if has_res:
        sp_ref, y_ref, dinv_ref, xprev_ref = refs[:4]
        rest = refs[4:]
    else:
        sp_ref, y_ref, dinv_ref = refs[:3]
        rest = refs[3:]
    if has_next:
        b_ref, g_ref, be_ref, wn_ref, outx_ref, outy_ref = rest
    else:
        b_ref, g_ref, be_ref, outx_ref = rest

    y = y_ref[...]
    s = sp_ref[0, :N, :] + sp_ref[1, :N, :]
    dinv = dinv_ref[...]
    t = dinv * (s + y) + b_ref[...]
    mu = jnp.mean(t, axis=0, keepdims=True)
    d = t - mu
    var = jnp.mean(d * d, axis=0, keepdims=True)
    x = g_ref[...] * d * lax.rsqrt(var + EPS) + be_ref[...]
    x = jnp.maximum(x, 0.0)
    if has_res:
        x = x + xprev_ref[...]
    outx_ref[...] = x
    if has_next:
        outy_ref[...] = dinv * jnp.dot(x, wn_ref[...],
                                       preferred_element_type=jnp.float32)


_mid_first = pl.pallas_call(
    functools.partial(_mid_body, False, True),
    out_shape=(jax.ShapeDtypeStruct((N, H), jnp.float32),
               jax.ShapeDtypeStruct((N, H), jnp.float32)),
)
_mid_res = pl.pallas_call(
    functools.partial(_mid_body, True, True),
    out_shape=(jax.ShapeDtypeStruct((N, H), jnp.float32),
               jax.ShapeDtypeStruct((N, H), jnp.float32)),
)
_mid_last = pl.pallas_call(
    functools.partial(_mid_body, True, False),
    out_shape=jax.ShapeDtypeStruct((N, H), jnp.float32),
)


def kernel(edge_index, emb, W1, b1, g1, be1, W2, b2, g2, be2, W3, b3, g3, be3):
    pad_dst = (jnp.arange(NPAD, dtype=jnp.int32) % (NP - N)) + N
    src_p = jnp.pad(edge_index[0], (0, NPAD)).reshape(NW, GPT, CH)
    dst_p = jnp.concatenate([edge_index[1], pad_dst]).reshape(NW, GPT, CH)
    b1r, g1r, be1r = b1.reshape(1, H), g1.reshape(1, H), be1.reshape(1, H)
    b2r, g2r, be2r = b2.reshape(1, H), g2.reshape(1, H), be2.reshape(1, H)
    b3r, g3r, be3r = b3.reshape(1, H), g3.reshape(1, H), be3.reshape(1, H)

    deg_kernel, seg_kernel = _sc_kernels()
    degp = deg_kernel(dst_p)[:, :N]
    y1, dinv = _pre_call(degp.T, emb, W1)
    s1 = seg_kernel(y1, src_p, dst_p)
    x2, y2 = _mid_first(s1, y1, dinv, b1r, g1r, be1r, W2)
    s2 = seg_kernel(y2, src_p, dst_p)
    x3, y3 = _mid_res(s2, y2, dinv, x2, b2r, g2r, be2r, W3)
    s3 = seg_kernel(y3, src_p, dst_p)
    return _mid_last(s3, y3, dinv, x3, b3r, g3r, be3r)


# speedup vs baseline: 1.2970x; 1.0034x over previous
import functools

import jax
import jax.numpy as jnp
from jax import lax
from jax.experimental import pallas as pl
from jax.experimental.pallas import tpu as pltpu
from jax.experimental.pallas import tpu_sc as plsc

N = 10000
H = 128
E = 320000
NC = 2
NS = 16
NW = NC * NS
CH = 128
GPT = 80
EPT = GPT * CH
EPAD = NW * EPT
NPAD = EPAD - E
NP = 10240
RPT = NP // NS
ZR = 32
SGC = 40
NSTG = GPT // SGC
NCHUNK = EPAD // CH
CA = 80
CB = NCHUNK // NS - CA
HH = H // 2
EPS = 1e-5


def _deg_body(dst_hbm, out_hbm, dst_v, ones_v, zb_v, acc):
    cid = lax.axis_index("c")
    sid = lax.axis_index("s")
    wid = sid * NC + cid
    one16 = jnp.ones((16,), jnp.float32)
    zero16 = jnp.zeros((16,), jnp.float32)
    for i in range(CH // 16):
        ones_v[pl.ds(i * 16, 16)] = one16
    for i in range(640 // 16):
        zb_v[pl.ds(i * 16, 16)] = zero16

    pltpu.sync_copy(zb_v, acc.at[pl.ds(sid * 640, 640)])
    plsc.subcore_barrier()

    pltpu.sync_copy(dst_hbm.at[wid], dst_v)

    def body(g, carry):
        pltpu.sync_copy(ones_v, acc.at[dst_v.at[g]], add=True)
        return carry

    lax.fori_loop(0, GPT, body, 0)
    plsc.subcore_barrier()

    pltpu.sync_copy(acc.at[pl.ds(sid * 640, 640)], zb_v)
    pltpu.sync_copy(zb_v, out_hbm.at[cid, pl.ds(sid * 640, 640)])


def _seg_body(y_hbm, src_hbm, dst_hbm, out_hbm,
              src_v, dst_v, rows_v, acc, sem_a, sem_b):
    cid = lax.axis_index("c")
    sid = lax.axis_index("s")
    zero16 = jnp.zeros((16,), jnp.float32)

    def zfill(r, carry):
        for c in range(H // 16):
            rows_v[0, r, pl.ds(c * 16, 16)] = zero16
        return carry

    lax.fori_loop(0, CH, zfill, 0)

    for k in range(RPT // CH):
        pltpu.async_copy(rows_v.at[0],
                         acc.at[pl.ds(sid * RPT + k * CH, CH)], sem_a)
    for k in range(RPT // CH):
        pltpu.make_async_copy(rows_v.at[0],
                              acc.at[pl.ds(sid * RPT + k * CH, CH)],
                              sem_a).wait()
    plsc.subcore_barrier()

    wid = sid * NC + cid

    def run_chunks(base, nstages):
        def stage_body(st, carry):
            pltpu.sync_copy(src_hbm.at[wid, pl.ds(st * SGC, SGC)], src_v)
            pltpu.sync_copy(dst_hbm.at[wid, pl.ds(st * SGC, SGC)], dst_v)
            pltpu.async_copy(y_hbm.at[src_v.at[0]], rows_v.at[0], sem_a)

            def body(j, c2):
                g0 = 2 * j
                pltpu.async_copy(y_hbm.at[src_v.at[g0 + 1]], rows_v.at[1],
                                 sem_b)
                pltpu.make_async_copy(y_hbm.at[src_v.at[g0]], rows_v.at[0],
                                      sem_a).wait()
                pltpu.sync_copy(rows_v.at[0], acc.at[dst_v.at[g0]], add=True)

                @pl.when(j < SGC // 2 - 1)
                def _():
                    pltpu.async_copy(y_hbm.at[src_v.at[g0 + 2]],
                                     rows_v.at[0], sem_a)

                pltpu.make_async_copy(y_hbm.at[src_v.at[g0 + 1]],
                                      rows_v.at[1], sem_b).wait()
                pltpu.sync_copy(rows_v.at[1], acc.at[dst_v.at[g0 + 1]],
                                add=True)
                return c2

            lax.fori_loop(0, SGC // 2, body, 0)
            return carry

        lax.fori_loop(0, nstages, stage_body, 0)

    run_chunks(0, NSTG)
    plsc.subcore_barrier()

    sems = (sem_a, sem_b)
    for k in range(RPT // CH):
        b = k % 2
        base = sid * RPT + k * CH
        if k >= 2:
            pbase = sid * RPT + (k - 2) * CH
            pltpu.make_async_copy(rows_v.at[b],
                                  out_hbm.at[cid, pl.ds(pbase, CH)],
                                  sems[b]).wait()
        pltpu.sync_copy(acc.at[pl.ds(base, CH)], rows_v.at[b])
        pltpu.async_copy(rows_v.at[b], out_hbm.at[cid, pl.ds(base, CH)],
                         sems[b])
    for k in range(RPT // CH - 2, RPT // CH):
        b = k % 2
        base = sid * RPT + k * CH
        pltpu.make_async_copy(rows_v.at[b],
                              out_hbm.at[cid, pl.ds(base, CH)],
                              sems[b]).wait()


@functools.lru_cache(maxsize=None)
def _sc_kernels():
    mesh = plsc.VectorSubcoreMesh(core_axis_name="c", subcore_axis_name="s",
                                  num_cores=NC, num_subcores=NS)
    deg_kernel = pl.kernel(
        _deg_body,
        out_type=jax.ShapeDtypeStruct((NC, NP), jnp.float32),
        mesh=mesh,
        scratch_types=[
            pltpu.VMEM((GPT, CH), jnp.int32),
            pltpu.VMEM((CH,), jnp.float32),
            pltpu.VMEM((640,), jnp.float32),
            pltpu.VMEM_SHARED((NP,), jnp.float32),
        ],
    )
    seg_kernel = pl.kernel(
        _seg_body,
        out_type=jax.ShapeDtypeStruct((NC, NP, H), jnp.float32),
        mesh=mesh,
        scratch_types=[
            pltpu.VMEM((SGC, CH), jnp.int32),
            pltpu.VMEM((SGC, CH), jnp.int32),
            pltpu.VMEM((2, CH, H), jnp.float32),
            pltpu.VMEM_SHARED((NP, H), jnp.float32),
            pltpu.SemaphoreType.DMA,
            pltpu.SemaphoreType.DMA,
        ],
    )
    return deg_kernel, seg_kernel


def _pre_body(degT_ref, x_ref, w_ref, y_ref, dinv_ref):
    dp = degT_ref[...]
    deg = dp[:, 0:1] + dp[:, 1:2] + 1.0
    dinv = lax.rsqrt(deg)
    dinv_ref[...] = dinv
    y_ref[...] = dinv * jnp.dot(x_ref[...], w_ref[...],
                                preferred_element_type=jnp.float32)


_pre_call = pl.pallas_call(
    _pre_body,
    out_shape=(jax.ShapeDtypeStruct((N, H), jnp.float32),
               jax.ShapeDtypeStruct((N, 1), jnp.float32)),
)


def _mid_body(has_res, has_next, *refs):
    if has_res:
        sp_ref, y_ref, dinv_ref, xprev_ref = refs[:4]
        rest = refs[4:]
    else:
        sp_ref, y_ref, dinv_ref = refs[:3]
        rest = refs[3:]
    if has_next:
        b_ref, g_ref, be_ref, wn_ref, outx_ref, outy_ref = rest
    else:
        b_ref, g_ref, be_ref, outx_ref = rest

    y = y_ref[...]
    s = sp_ref[0, :N, :] + sp_ref[1, :N, :]
    dinv = dinv_ref[...]
    t = dinv * (s + y) + b_ref[...]
    mu = jnp.mean(t, axis=0, keepdims=True)
    d = t - mu
    var = jnp.mean(d * d, axis=0, keepdims=True)
    x = g_ref[...] * d * lax.rsqrt(var + EPS) + be_ref[...]
    x = jnp.maximum(x, 0.0)
    if has_res:
        x = x + xprev_ref[...]
    outx_ref[...] = x
    if has_next:
        outy_ref[...] = dinv * jnp.dot(x, wn_ref[...],
                                       preferred_element_type=jnp.float32)


_mid_first = pl.pallas_call(
    functools.partial(_mid_body, False, True),
    out_shape=(jax.ShapeDtypeStruct((N, H), jnp.float32),
               jax.ShapeDtypeStruct((N, H), jnp.float32)),
)
_mid_res = pl.pallas_call(
    functools.partial(_mid_body, True, True),
    out_shape=(jax.ShapeDtypeStruct((N, H), jnp.float32),
               jax.ShapeDtypeStruct((N, H), jnp.float32)),
)
_mid_last = pl.pallas_call(
    functools.partial(_mid_body, True, False),
    out_shape=jax.ShapeDtypeStruct((N, H), jnp.float32),
)


def kernel(edge_index, emb, W1, b1, g1, be1, W2, b2, g2, be2, W3, b3, g3, be3):
    pad_dst = (jnp.arange(NPAD, dtype=jnp.int32) % (NP - N)) + N
    src_p = jnp.pad(edge_index[0], (0, NPAD)).reshape(NW, GPT, CH)
    dst_p = jnp.concatenate([edge_index[1], pad_dst]).reshape(NW, GPT, CH)
    b1r, g1r, be1r = b1.reshape(1, H), g1.reshape(1, H), be1.reshape(1, H)
    b2r, g2r, be2r = b2.reshape(1, H), g2.reshape(1, H), be2.reshape(1, H)
    b3r, g3r, be3r = b3.reshape(1, H), g3.reshape(1, H), be3.reshape(1, H)

    deg_kernel, seg_kernel = _sc_kernels()
    degp = deg_kernel(dst_p)[:, :N]
    y1, dinv = _pre_call(degp.T, emb, W1)
    s1 = seg_kernel(y1, src_p, dst_p)
    x2, y2 = _mid_first(s1, y1, dinv, b1r, g1r, be1r, W2)
    s2 = seg_kernel(y2, src_p, dst_p)
    x3, y3 = _mid_res(s2, y2, dinv, x2, b2r, g2r, be2r, W3)
    s3 = seg_kernel(y3, src_p, dst_p)
    return _mid_last(s3, y3, dinv, x3, b3r, g3r, be3r)
